# baseline (speedup 1.0000x reference)
import functools

import jax
import jax.numpy as jnp
from jax import lax
from jax.experimental import pallas as pl
from jax.experimental.pallas import tpu as pltpu
from jax.experimental.pallas import tpu_sc as plsc

B, N, D = 8, 32768, 64
R = B * D
NC, NS = 2, 16
NW = NC * NS
RPW = R // NW
RG = 8
CN = 4096
NP = N // CN
VECS = CN // 16


def _sc_body(rows_hbm, mask_hbm, out_hbm, buf, mbuf, insems, outsems, msems):
    cid = lax.axis_index("c")
    sid = lax.axis_index("s")
    wid = sid * NC + cid
    r0 = wid * RPW

    def slices(c):
        rbase = pl.multiple_of(r0 + (c % 2) * RG, 8)
        nbase = pl.multiple_of((c // 2) * CN, 8)
        return pl.ds(rbase, RG), pl.ds(nbase, CN)

    def fire_in(c, s):
        ri, ni = slices(c)
        pltpu.async_copy(rows_hbm.at[ri, ni], buf.at[s], insems[s])

    def wait_in(s):
        pltpu.make_async_copy(
            rows_hbm.at[pl.ds(0, RG), pl.ds(0, CN)], buf.at[s], insems[s]
        ).wait()

    def fire_out(c, s):
        ri, ni = slices(c)
        pltpu.async_copy(buf.at[s], out_hbm.at[ri, ni], outsems[s])

    def wait_out(s):
        pltpu.make_async_copy(
            buf.at[s], out_hbm.at[pl.ds(0, RG), pl.ds(0, CN)], outsems[s]
        ).wait()

    def fire_mask(p, m):
        nbase = pl.multiple_of(p * CN, 8)
        pltpu.async_copy(mask_hbm.at[pl.ds(nbase, CN)], mbuf.at[m], msems[m])

    def wait_mask(m):
        pltpu.make_async_copy(
            mask_hbm.at[pl.ds(0, CN)], mbuf.at[m], msems[m]
        ).wait()

    def compute(s, m):
        @pl.loop(0, VECS)
        def vec_loop(v):
            o = v * 16
            keep = mbuf[m, pl.ds(o, 16)] != 0
            xs = [buf[s, r, pl.ds(o, 16)] for r in range(RG)]
            for r in range(RG):
                buf[s, r, pl.ds(o, 16)] = jnp.where(keep, xs[r], 0.0)

    fire_in(0, 0)
    fire_in(1, 1)
    fire_mask(0, 0)
    for p in range(NP):
        c0 = 2 * p
        m = p % 2
        wait_mask(m)
        wait_in(0)
        compute(0, m)
        fire_out(c0, 0)
        wait_in(1)
        compute(1, m)
        fire_out(c0 + 1, 1)
        if p < NP - 1:
            fire_mask(p + 1, (p + 1) % 2)
            wait_out(0)
            fire_in(c0 + 2, 0)
            wait_out(1)
            fire_in(c0 + 3, 1)
    wait_out(0)
    wait_out(1)


_sc_kernel = functools.partial(
    pl.kernel,
    out_type=jax.ShapeDtypeStruct((R, N), jnp.float32),
    mesh=plsc.VectorSubcoreMesh(core_axis_name="c", subcore_axis_name="s"),
    scratch_types=[
        pltpu.VMEM((2, RG, CN), jnp.float32),
        pltpu.VMEM((2, CN), jnp.int32),
        [pltpu.SemaphoreType.DMA, pltpu.SemaphoreType.DMA],
        [pltpu.SemaphoreType.DMA, pltpu.SemaphoreType.DMA],
        [pltpu.SemaphoreType.DMA, pltpu.SemaphoreType.DMA],
    ],
)(_sc_body)


def kernel(data, mask_array):
    mask_i = mask_array.astype(jnp.int32)
    rows = jnp.transpose(data, (0, 2, 1)).reshape(R, N)
    out2 = _sc_kernel(rows, mask_i)
    return jnp.transpose(out2.reshape(B, D, N), (2, 0, 1))


# speedup vs baseline: 1.0044x; 1.0044x over previous
import functools

import jax
import jax.numpy as jnp
from jax import lax
from jax.experimental import pallas as pl
from jax.experimental.pallas import tpu as pltpu
from jax.experimental.pallas import tpu_sc as plsc

B, N, D = 8, 32768, 64
R = B * D
NC, NS = 2, 16
NW = NC * NS
RPW = R // NW
RG = 8
CN = 2048
NCH = (RPW // RG) * (N // CN)
NSL = 4
VECS = CN // 16


def _sc_body(rows_hbm, mask_hbm, out_hbm, buf, mbuf, insems, msems, outsems):
    cid = lax.axis_index("c")
    sid = lax.axis_index("s")
    wid = sid * NC + cid
    r0 = wid * RPW

    def slices(c):
        rbase = pl.multiple_of(r0 + (c % 2) * RG, 8)
        nbase = pl.multiple_of((c // 2) * CN, 8)
        return pl.ds(rbase, RG), pl.ds(nbase, CN)

    def fire_in(c, s):
        ri, ni = slices(c)
        pltpu.async_copy(rows_hbm.at[ri, ni], buf.at[s], insems[s])
        pltpu.async_copy(mask_hbm.at[ni], mbuf.at[s], msems[s])

    def wait_in(s):
        pltpu.make_async_copy(
            rows_hbm.at[pl.ds(0, RG), pl.ds(0, CN)], buf.at[s], insems[s]
        ).wait()
        pltpu.make_async_copy(
            mask_hbm.at[pl.ds(0, CN)], mbuf.at[s], msems[s]
        ).wait()

    def fire_out(c, s):
        ri, ni = slices(c)
        pltpu.async_copy(buf.at[s], out_hbm.at[ri, ni], outsems[s])

    def wait_out(s):
        pltpu.make_async_copy(
            buf.at[s], out_hbm.at[pl.ds(0, RG), pl.ds(0, CN)], outsems[s]
        ).wait()

    def compute(s):
        @pl.loop(0, VECS)
        def vec_loop(v):
            o = v * 16
            keep = mbuf[s, pl.ds(o, 16)] != 0
            xs = [buf[s, r, pl.ds(o, 16)] for r in range(RG)]
            for r in range(RG):
                buf[s, r, pl.ds(o, 16)] = jnp.where(keep, xs[r], 0.0)

    def step(c, k, first, last):
        s2 = (k + 2) % NSL
        wait_in(k)
        if not first:
            wait_out(s2)
        if not last:
            fire_in(c + 2, s2)
        compute(k)
        fire_out(c, k)

    fire_in(0, 0)
    fire_in(1, 1)
    step(0, 0, True, False)
    step(1, 1, True, False)
    step(2, 2, False, False)
    step(3, 3, False, False)

    @pl.loop(1, NCH // NSL - 1)
    def lp(g):
        c0 = NSL * g
        step(c0 + 0, 0, False, False)
        step(c0 + 1, 1, False, False)
        step(c0 + 2, 2, False, False)
        step(c0 + 3, 3, False, False)

    c0 = NCH - NSL
    step(c0 + 0, 0, False, False)
    step(c0 + 1, 1, False, False)
    step(c0 + 2, 2, False, True)
    step(c0 + 3, 3, False, True)
    wait_out(2)
    wait_out(3)


_sc_kernel = functools.partial(
    pl.kernel,
    out_type=jax.ShapeDtypeStruct((R, N), jnp.float32),
    mesh=plsc.VectorSubcoreMesh(core_axis_name="c", subcore_axis_name="s"),
    scratch_types=[
        pltpu.VMEM((NSL, RG, CN), jnp.float32),
        pltpu.VMEM((NSL, CN), jnp.int32),
        [pltpu.SemaphoreType.DMA] * NSL,
        [pltpu.SemaphoreType.DMA] * NSL,
        [pltpu.SemaphoreType.DMA] * NSL,
    ],
)(_sc_body)


def kernel(data, mask_array):
    mask_i = mask_array.astype(jnp.int32)
    rows = jnp.transpose(data, (0, 2, 1)).reshape(R, N)
    out2 = _sc_kernel(rows, mask_i)
    return jnp.transpose(out2.reshape(B, D, N), (2, 0, 1))


# speedup vs baseline: 1.2211x; 1.2158x over previous
import functools

import jax
import jax.numpy as jnp
from jax import lax
from jax.experimental import pallas as pl
from jax.experimental.pallas import tpu as pltpu
from jax.experimental.pallas import tpu_sc as plsc

B, N, D = 8, 32768, 64
R = B * D
NC, NS = 2, 16
NW = NC * NS
RPW = R // NW
RG = RPW
CN = 1024
NCH = (RPW // RG) * (N // CN)
NSL = 4
VECS = CN // 16


def _sc_body(rows_hbm, mask_hbm, out_hbm, buf, mbuf, insems, msems, outsems):
    cid = lax.axis_index("c")
    sid = lax.axis_index("s")
    wid = sid * NC + cid
    r0 = wid * RPW

    def slices(c):
        rbase = pl.multiple_of(r0, 8)
        nbase = pl.multiple_of(c * CN, 8)
        return pl.ds(rbase, RG), pl.ds(nbase, CN)

    def fire_in(c, s):
        ri, ni = slices(c)
        pltpu.async_copy(rows_hbm.at[ri, ni], buf.at[s], insems[s])
        pltpu.async_copy(mask_hbm.at[ni], mbuf.at[s], msems[s])

    def wait_in(s):
        pltpu.make_async_copy(
            rows_hbm.at[pl.ds(0, RG), pl.ds(0, CN)], buf.at[s], insems[s]
        ).wait()
        pltpu.make_async_copy(
            mask_hbm.at[pl.ds(0, CN)], mbuf.at[s], msems[s]
        ).wait()

    def fire_out(c, s):
        ri, ni = slices(c)
        pltpu.async_copy(buf.at[s], out_hbm.at[ri, ni], outsems[s])

    def wait_out(s):
        pltpu.make_async_copy(
            buf.at[s], out_hbm.at[pl.ds(0, RG), pl.ds(0, CN)], outsems[s]
        ).wait()

    def compute(s):
        @pl.loop(0, VECS)
        def vec_loop(v):
            o = v * 16
            keep = mbuf[s, pl.ds(o, 16)] != 0
            xs = [buf[s, r, pl.ds(o, 16)] for r in range(RG)]
            for r in range(RG):
                buf[s, r, pl.ds(o, 16)] = jnp.where(keep, xs[r], 0.0)

    def step(c, k, first, last):
        s2 = (k + 2) % NSL
        wait_in(k)
        if not first:
            wait_out(s2)
        if not last:
            fire_in(c + 2, s2)
        compute(k)
        fire_out(c, k)

    fire_in(0, 0)
    fire_in(1, 1)
    step(0, 0, True, False)
    step(1, 1, True, False)
    step(2, 2, False, False)
    step(3, 3, False, False)

    @pl.loop(1, NCH // NSL - 1)
    def lp(g):
        c0 = NSL * g
        step(c0 + 0, 0, False, False)
        step(c0 + 1, 1, False, False)
        step(c0 + 2, 2, False, False)
        step(c0 + 3, 3, False, False)

    c0 = NCH - NSL
    step(c0 + 0, 0, False, False)
    step(c0 + 1, 1, False, False)
    step(c0 + 2, 2, False, True)
    step(c0 + 3, 3, False, True)
    wait_out(2)
    wait_out(3)


_sc_kernel = functools.partial(
    pl.kernel,
    out_type=jax.ShapeDtypeStruct((R, N), jnp.float32),
    mesh=plsc.VectorSubcoreMesh(core_axis_name="c", subcore_axis_name="s"),
    scratch_types=[
        pltpu.VMEM((NSL, RG, CN), jnp.float32),
        pltpu.VMEM((NSL, CN), jnp.int32),
        [pltpu.SemaphoreType.DMA] * NSL,
        [pltpu.SemaphoreType.DMA] * NSL,
        [pltpu.SemaphoreType.DMA] * NSL,
    ],
)(_sc_body)


def kernel(data, mask_array):
    mask_i = mask_array.astype(jnp.int32)
    rows = jnp.transpose(data, (0, 2, 1)).reshape(R, N)
    out2 = _sc_kernel(rows, mask_i)
    return jnp.transpose(out2.reshape(B, D, N), (2, 0, 1))
